# Initial kernel scaffold; baseline (speedup 1.0000x reference)
import jax
import jax.numpy as jnp
from jax.experimental import pallas as pl


def kernel(gt_sensor2_T_sensor1, gt_e_l, pred_e_l, pred_f_score):
    raise NotImplementedError("write your pallas kernel here")


# speedup vs baseline: 2.3979x; 2.3979x over previous
from math import pi

import jax
import jax.numpy as jnp
from jax import lax
from jax.experimental import pallas as pl

_B = 16384
_W = 1024
_POSITIVE_NUM = 8
_NUM_NEG = 24
_LAMBDA_FOV = 1.0

_LO_BITS = 953267991
_HI_BITS = 1065353216
_N_ITERS = 27

_BR = 512


def _c3(ref, i, j):
    c = 3 * i + j
    return ref[:, c:c + 1]


def _bf(x):
    return x.astype(jnp.bfloat16).astype(jnp.float32)


def _fused_kernel(ri_ref, ge_ref, pe_ref, p_ref, fsg_ref, fl_ref, loss_ref):
    f32 = jnp.float32

    ri = [[_bf(_c3(ri_ref, i, j)) for j in range(3)] for i in range(3)]
    peb = [[_bf(_c3(pe_ref, i, j)) for j in range(3)] for i in range(3)]
    geb = [[_bf(_c3(ge_ref, i, j)) for j in range(3)] for i in range(3)]

    def mm_entry(x, y, i, j):
        return (x[i][0] * y[0][j] + x[i][1] * y[1][j]) + x[i][2] * y[2][j]

    ax = mm_entry(peb, ri, 0, 0)
    ay = mm_entry(peb, ri, 1, 0)
    yaw = jnp.arctan2(ay, ax)
    f_idx = (-yaw + pi) / (2.0 * pi) * _W
    xmin = f_idx.astype(jnp.int32) - _POSITIVE_NUM // 2

    m00, m01, m02 = (mm_entry(geb, ri, 0, j) for j in range(3))
    m10, m11, m12 = (mm_entry(geb, ri, 1, j) for j in range(3))
    m20, m21, m22 = (mm_entry(geb, ri, 2, j) for j in range(3))

    a00 = m11 * m22 - m12 * m21
    a01 = m02 * m21 - m01 * m22
    a02 = m01 * m12 - m02 * m11
    a10 = m12 * m20 - m10 * m22
    a11 = m00 * m22 - m02 * m20
    a12 = m02 * m10 - m00 * m12
    a20 = m10 * m21 - m11 * m20
    a21 = m01 * m20 - m00 * m21
    a22 = m00 * m11 - m01 * m10
    det = m00 * a00 + m01 * a10 + m02 * a20
    idet = 1.0 / det

    z = jnp.zeros_like(a00)
    o = jnp.ones_like(a00)
    fl_ref[...] = jnp.concatenate(
        [a00 * idet, a01 * idet, a02 * idet, z,
         a10 * idet, a11 * idet, a12 * idet, z,
         a20 * idet, a21 * idet, a22 * idet, z,
         z, z, z, o], axis=1)

    colid = lax.broadcasted_iota(jnp.int32, (_BR, _W), 1)
    rel = (colid - xmin) & (_W - 1)
    pos = rel < _POSITIVE_NUM
    fsg_ref[...] = pos.astype(f32)

    p = p_ref[...]
    pt = jnp.where(pos, f32(0.0), p)
    bits = lax.bitcast_convert_type(pt, jnp.int32)

    lo = jnp.full((_BR, 1), _LO_BITS, jnp.int32)
    hi = jnp.full((_BR, 1), _HI_BITS, jnp.int32)
    for _ in range(_N_ITERS):
        mid = lo + ((hi - lo + 1) >> 1)
        cnt = jnp.sum((bits >= mid).astype(jnp.int32), axis=1, keepdims=True)
        ok = cnt >= _NUM_NEG
        lo = jnp.where(ok, mid, lo)
        hi = jnp.where(ok, hi, mid - 1)

    kbits = lo
    kval = lax.bitcast_convert_type(kbits, f32)

    ln1mp = jnp.log1p(-p)
    sel = bits > kbits
    ngt = jnp.sum(jnp.where(sel, f32(1.0), f32(0.0)), axis=1, keepdims=True)
    sneg = jnp.sum(jnp.where(sel, ln1mp, f32(0.0)), axis=1, keepdims=True)
    neg_part = -(sneg + (_NUM_NEG - ngt) * jnp.log1p(-kval))

    q = jnp.where(pos, p, f32(1.0))
    w = _W
    while w > 1:
        w //= 2
        q = q[:, :w] * q[:, w:2 * w]
    pos_part = -jnp.log(q)

    loss_ref[...] = jnp.sum(pos_part + neg_part, keepdims=True)[None]


def kernel(gt_sensor2_T_sensor1, gt_e_l, pred_e_l, pred_f_score):
    ri = jnp.linalg.inv(gt_sensor2_T_sensor1[:, :3, :3]).reshape(_B, 9)
    ge = gt_e_l[:, :3, :3].reshape(_B, 9)
    pe = pred_e_l[:, :3, :3].reshape(_B, 9)
    grid = _B // _BR

    fsg, fl, partial = pl.pallas_call(
        _fused_kernel,
        grid=(grid,),
        in_specs=[
            pl.BlockSpec((_BR, 9), lambda g: (g, 0)),
            pl.BlockSpec((_BR, 9), lambda g: (g, 0)),
            pl.BlockSpec((_BR, 9), lambda g: (g, 0)),
            pl.BlockSpec((_BR, _W), lambda g: (g, 0)),
        ],
        out_specs=[
            pl.BlockSpec((_BR, _W), lambda g: (g, 0)),
            pl.BlockSpec((_BR, 16), lambda g: (g, 0)),
            pl.BlockSpec((1, 1, 1), lambda g: (g, 0, 0)),
        ],
        out_shape=[
            jax.ShapeDtypeStruct((_B, _W), jnp.float32),
            jax.ShapeDtypeStruct((_B, 16), jnp.float32),
            jax.ShapeDtypeStruct((grid, 1, 1), jnp.float32),
        ],
    )(ri, ge, pe, pred_f_score)

    denom = jnp.float32(_B * (_POSITIVE_NUM + _NUM_NEG))
    loss = jnp.sum(partial) / denom * _LAMBDA_FOV
    return (loss, fsg, fl.reshape(_B, 4, 4))
